# Initial kernel scaffold; baseline (speedup 1.0000x reference)
import jax
import jax.numpy as jnp
from jax.experimental import pallas as pl


def kernel(alpha, ctc_log_probs, mask, text_length):
    raise NotImplementedError("write your pallas kernel here")


# speedup vs baseline: 1.1896x; 1.1896x over previous
import math

import jax
import jax.numpy as jnp
from jax import lax
from jax.experimental import pallas as pl
from jax.experimental.pallas import tpu as pltpu
from jax.experimental.pallas import tpu_sc as plsc

_B = 16
_T = 2048
_V = 512
_L = 16
_NCHUNK = _T // _L
_K = 208
_KCAP = 224
_NK = _K // _L
_THR = math.log(1.0 - 0.7)


def _sc_body(alpha_hbm, ctc_hbm, mask_hbm, tl_hbm, idx_hbm, out_hbm,
             idx_v, blank_v, a_v, m_v, cc_v, ac_v, tl_v, row_v, sem):
    c = lax.axis_index("c")
    s = lax.axis_index("s")

    @pl.when(c == 0)
    def _work():
        w = s

        pltpu.sync_copy(idx_hbm.at[w], idx_v)
        pltpu.sync_copy(alpha_hbm.at[w], a_v)
        pltpu.sync_copy(mask_hbm.at[w], m_v)
        pltpu.sync_copy(tl_hbm, tl_v)

        copies = [
            pltpu.make_async_copy(
                ctc_hbm.at[idx_v.at[j]],
                blank_v.at[pl.ds(j * 128, 128)],
                sem,
            )
            for j in range(16)
        ]
        for cp in copies:
            cp.start()
        for cp in copies:
            cp.wait()

        def zinit(i, _):
            cc_v[pl.ds(i * _L, _L)] = jnp.zeros((_L,), jnp.float32)
            ac_v[pl.ds(i * _L, _L)] = jnp.zeros((_L,), jnp.float32)
            return 0
        lax.fori_loop(0, _KCAP // _L, zinit, 0)

        def chunk(i, carry):
            csum, cnt, isum = carry
            t0 = i * _L
            av = a_v[pl.ds(t0, _L)]
            bv = blank_v[pl.ds(t0, _L)]
            mv = m_v[pl.ds(t0, _L)]
            spike = (bv < _THR) & (mv != 0.0)
            cs = plsc.cumsum(av) + csum
            spk_i = spike.astype(jnp.int32)
            rank = plsc.cumsum(spk_i)
            pos = cnt + rank - 1
            wmask = spike & (pos < jnp.int32(_KCAP - _L + 8))
            plsc.store_scatter(cc_v, [pos], cs, mask=wmask)
            plsc.store_scatter(ac_v, [pos], av, mask=wmask)
            tvec = t0 + lax.iota(jnp.int32, _L)
            isum = isum + jnp.sum(jnp.where(spike, tvec, 0))
            cnt = cnt + jnp.sum(spk_i)
            csum = csum + jnp.sum(av)
            return csum, cnt, isum

        _, cnt, isum = lax.fori_loop(
            0, _NCHUNK, chunk,
            (jnp.float32(0.0), jnp.int32(0), jnp.int32(0)))

        tlw = plsc.load_gather(tl_v, [jnp.full((_L,), w, jnp.int32)])
        for kc in range(_NK):
            k0 = kc * _L
            kvec = k0 + lax.iota(jnp.int32, _L)
            c0 = cc_v[pl.ds(k0, _L)]
            c1 = cc_v[pl.ds(k0 + 1, _L)]
            a0 = ac_v[pl.ds(k0, _L)]
            bd = c1 - c0 + a0
            valid = kvec < (cnt - 1)
            term = jnp.where(valid, jnp.abs(bd - 1.0), 1.0)
            term = jnp.where(kvec < tlw, term, 0.0)
            row_v[pl.ds(k0, _L)] = term

        n_w = jnp.where(isum > 0, cnt - 1, 1)
        row_v[pl.ds(_K, _L)] = jnp.full((_L,), n_w, jnp.int32).astype(jnp.float32)
        pltpu.sync_copy(row_v, out_hbm.at[w])


def _tc_finalize(rows_ref, out_ref):
    x = rows_ref[...]
    maxn = jnp.max(x[:, _K:])
    kvec = lax.broadcasted_iota(jnp.int32, (_B, _KCAP), 1).astype(jnp.float32)
    m = (kvec < maxn) & (kvec < float(_K))
    out_ref[0, 0] = jnp.sum(jnp.where(m, x, 0.0)) * (1.0 / _B)


def kernel(alpha, ctc_log_probs, mask, text_length):
    ctc_flat = ctc_log_probs.reshape(-1)
    t = jnp.arange(_T, dtype=jnp.int32)
    b = jnp.arange(_B, dtype=jnp.int32)
    idx = ((b[:, None] * _T + t[None, :]) * _V).reshape(_B, 16, 128)

    mesh = plsc.VectorSubcoreMesh(core_axis_name="c", subcore_axis_name="s")
    sc_run = pl.kernel(
        _sc_body,
        out_type=jax.ShapeDtypeStruct((_B, _KCAP), jnp.float32),
        mesh=mesh,
        compiler_params=pltpu.CompilerParams(needs_layout_passes=False),
        scratch_types=[
            pltpu.VMEM((16, 128), jnp.int32),
            pltpu.VMEM((_T,), jnp.float32),
            pltpu.VMEM((_T,), jnp.float32),
            pltpu.VMEM((_T,), jnp.float32),
            pltpu.VMEM((_KCAP,), jnp.float32),
            pltpu.VMEM((_KCAP,), jnp.float32),
            pltpu.VMEM((_L,), jnp.int32),
            pltpu.VMEM((_KCAP,), jnp.float32),
            pltpu.SemaphoreType.DMA,
        ],
    )
    rows = sc_run(alpha, ctc_flat, mask, text_length.astype(jnp.int32), idx)

    out = pl.pallas_call(
        _tc_finalize,
        out_shape=jax.ShapeDtypeStruct((1, 1), jnp.float32),
        out_specs=pl.BlockSpec(memory_space=pltpu.SMEM),
    )(rows)
    return out[0, 0]


# speedup vs baseline: 1.7830x; 1.4988x over previous
import math

import jax
import jax.numpy as jnp
from jax import lax
from jax.experimental import pallas as pl
from jax.experimental.pallas import tpu as pltpu
from jax.experimental.pallas import tpu_sc as plsc

_B = 16
_T = 2048
_V = 512
_L = 16
_NCHUNK = _T // _L
_K = 208
_KCAP = 224
_NK = _K // _L
_THR = math.log(1.0 - 0.7)


def _tc_prep(ctc_ref, alpha_ref, mask_ref, spike_ref, csum_ref):
    blank = ctc_ref[0, :, 0].reshape(1, 1, _T)
    spike = (blank < _THR) & (mask_ref[...] != 0.0)
    spike_ref[...] = spike.astype(jnp.float32)
    x = alpha_ref[...]
    sh = 1
    while sh < _T:
        x = x + jnp.concatenate(
            [jnp.zeros((1, 1, sh), jnp.float32), x[:, :, :-sh]], axis=2)
        sh *= 2
    csum_ref[...] = x


def _sc_body(spike_hbm, csum_hbm, alpha_hbm, tl_hbm, out_hbm,
             s_v, c_v, a_v, cc_v, ac_v, tl_v, row_v):
    c = lax.axis_index("c")
    s = lax.axis_index("s")

    @pl.when(c == 0)
    def _work():
        w = s
        pltpu.sync_copy(spike_hbm.at[w], s_v)
        pltpu.sync_copy(csum_hbm.at[w], c_v)
        pltpu.sync_copy(alpha_hbm.at[w], a_v)
        pltpu.sync_copy(tl_hbm, tl_v)

        def zinit(i, _):
            cc_v[pl.ds(i * _L, _L)] = jnp.zeros((_L,), jnp.float32)
            ac_v[pl.ds(i * _L, _L)] = jnp.zeros((_L,), jnp.float32)
            return 0
        lax.fori_loop(0, _KCAP // _L, zinit, 0)

        def chunk(i, carry):
            cnt, ivec = carry
            t0 = i * _L
            spike = s_v[pl.ds(t0, _L)] != 0.0
            spk_i = spike.astype(jnp.int32)
            rank = plsc.cumsum(spk_i)
            pos = cnt + rank - 1
            wmask = spike & (pos < jnp.int32(_KCAP - _L + 8))
            plsc.store_scatter(cc_v, [pos], c_v[pl.ds(t0, _L)], mask=wmask)
            plsc.store_scatter(ac_v, [pos], a_v[pl.ds(t0, _L)], mask=wmask)
            tvec = t0 + lax.iota(jnp.int32, _L)
            ivec = ivec + jnp.where(spike, tvec, 0)
            cnt = cnt + rank[_L - 1]
            return cnt, ivec

        cnt, ivec = lax.fori_loop(
            0, _NCHUNK, chunk,
            (jnp.int32(0), jnp.zeros((_L,), jnp.int32)))
        isum = jnp.sum(ivec)

        tlw = plsc.load_gather(tl_v, [jnp.full((_L,), w, jnp.int32)])
        for kc in range(_NK):
            k0 = kc * _L
            kvec = k0 + lax.iota(jnp.int32, _L)
            c0 = cc_v[pl.ds(k0, _L)]
            c1 = cc_v[pl.ds(k0 + 1, _L)]
            a0 = ac_v[pl.ds(k0, _L)]
            bd = c1 - c0 + a0
            valid = kvec < (cnt - 1)
            term = jnp.where(valid, jnp.abs(bd - 1.0), 1.0)
            term = jnp.where(kvec < tlw, term, 0.0)
            row_v[pl.ds(k0, _L)] = term

        n_w = jnp.where(isum > 0, cnt - 1, 1)
        row_v[pl.ds(_K, _L)] = jnp.full((_L,), n_w, jnp.int32).astype(jnp.float32)
        pltpu.sync_copy(row_v, out_hbm.at[w])


def _tc_finalize(rows_ref, out_ref):
    x = rows_ref[...]
    maxn = jnp.max(x[:, _K:])
    kvec = lax.broadcasted_iota(jnp.int32, (_B, _KCAP), 1).astype(jnp.float32)
    m = (kvec < maxn) & (kvec < float(_K))
    out_ref[0, 0] = jnp.sum(jnp.where(m, x, 0.0)) * (1.0 / _B)


def kernel(alpha, ctc_log_probs, mask, text_length):
    spike, csum = pl.pallas_call(
        _tc_prep,
        grid=(_B,),
        in_specs=[
            pl.BlockSpec((1, _T, 128), lambda b: (b, 0, 0)),
            pl.BlockSpec((1, 1, _T), lambda b: (b, 0, 0)),
            pl.BlockSpec((1, 1, _T), lambda b: (b, 0, 0)),
        ],
        out_specs=[
            pl.BlockSpec((1, 1, _T), lambda b: (b, 0, 0)),
            pl.BlockSpec((1, 1, _T), lambda b: (b, 0, 0)),
        ],
        out_shape=[
            jax.ShapeDtypeStruct((_B, 1, _T), jnp.float32),
            jax.ShapeDtypeStruct((_B, 1, _T), jnp.float32),
        ],
    )(ctc_log_probs, alpha.reshape(_B, 1, _T), mask.reshape(_B, 1, _T))
    spike = spike.reshape(_B, _T)
    csum = csum.reshape(_B, _T)

    mesh = plsc.VectorSubcoreMesh(core_axis_name="c", subcore_axis_name="s")
    sc_run = pl.kernel(
        _sc_body,
        out_type=jax.ShapeDtypeStruct((_B, _KCAP), jnp.float32),
        mesh=mesh,
        compiler_params=pltpu.CompilerParams(needs_layout_passes=False),
        scratch_types=[
            pltpu.VMEM((_T,), jnp.float32),
            pltpu.VMEM((_T,), jnp.float32),
            pltpu.VMEM((_T,), jnp.float32),
            pltpu.VMEM((_KCAP,), jnp.float32),
            pltpu.VMEM((_KCAP,), jnp.float32),
            pltpu.VMEM((_L,), jnp.int32),
            pltpu.VMEM((_KCAP,), jnp.float32),
        ],
    )
    rows = sc_run(spike, csum, alpha, text_length.astype(jnp.int32))

    out = pl.pallas_call(
        _tc_finalize,
        out_shape=jax.ShapeDtypeStruct((1, 1), jnp.float32),
        out_specs=pl.BlockSpec(memory_space=pltpu.SMEM),
    )(rows)
    return out[0, 0]


# speedup vs baseline: 1.9658x; 1.1025x over previous
import math

import jax
import jax.numpy as jnp
from jax import lax
from jax.experimental import pallas as pl
from jax.experimental.pallas import tpu as pltpu
from jax.experimental.pallas import tpu_sc as plsc

_B = 16
_T = 2048
_V = 512
_L = 16
_NCHUNK = _T // _L
_K = 208
_KCAP = 224
_NK = _K // _L
_THR = math.log(1.0 - 0.7)


def _tc_prep(ctc_ref, alpha_ref, mask_ref, spike_ref, csum_ref):
    x2d = ctc_ref[0]
    e0 = (lax.broadcasted_iota(jnp.int32, (1, 128), 1) == 0).astype(jnp.float32)
    blank = lax.dot_general(
        e0, x2d, (((1,), (1,)), ((), ())),
        precision=lax.Precision.HIGHEST).reshape(1, 1, _T)
    spike = (blank < _THR) & (mask_ref[...] != 0.0)
    spike_ref[...] = spike.astype(jnp.float32)
    x = alpha_ref[...]
    sh = 1
    while sh < _T:
        x = x + jnp.concatenate(
            [jnp.zeros((1, 1, sh), jnp.float32), x[:, :, :-sh]], axis=2)
        sh *= 2
    csum_ref[...] = x


def _sc_body(spike_hbm, csum_hbm, alpha_hbm, tl_hbm, out_hbm,
             s_v, c_v, a_v, cc_v, ac_v, tl_v, row_v):
    c = lax.axis_index("c")
    s = lax.axis_index("s")

    @pl.when(c == 0)
    def _work():
        w = s
        pltpu.sync_copy(spike_hbm.at[w], s_v)
        pltpu.sync_copy(csum_hbm.at[w], c_v)
        pltpu.sync_copy(alpha_hbm.at[w], a_v)
        pltpu.sync_copy(tl_hbm, tl_v)

        def zinit(i, _):
            cc_v[pl.ds(i * _L, _L)] = jnp.zeros((_L,), jnp.float32)
            ac_v[pl.ds(i * _L, _L)] = jnp.zeros((_L,), jnp.float32)
            return 0
        lax.fori_loop(0, _KCAP // _L, zinit, 0)

        def chunk(i, carry):
            cnt, ivec = carry
            t0 = i * _L
            spike = s_v[pl.ds(t0, _L)] != 0.0
            spk_i = spike.astype(jnp.int32)
            rank = plsc.cumsum(spk_i)
            pos = cnt + rank - 1
            wmask = spike & (pos < jnp.int32(_KCAP - _L + 8))
            plsc.store_scatter(cc_v, [pos], c_v[pl.ds(t0, _L)], mask=wmask)
            plsc.store_scatter(ac_v, [pos], a_v[pl.ds(t0, _L)], mask=wmask)
            tvec = t0 + lax.iota(jnp.int32, _L)
            ivec = ivec + jnp.where(spike, tvec, 0)
            cnt = cnt + rank[_L - 1]
            return cnt, ivec

        cnt, ivec = lax.fori_loop(
            0, _NCHUNK, chunk,
            (jnp.int32(0), jnp.zeros((_L,), jnp.int32)))
        isum = jnp.sum(ivec)

        tlw = plsc.load_gather(tl_v, [jnp.full((_L,), w, jnp.int32)])
        for kc in range(_NK):
            k0 = kc * _L
            kvec = k0 + lax.iota(jnp.int32, _L)
            c0 = cc_v[pl.ds(k0, _L)]
            c1 = cc_v[pl.ds(k0 + 1, _L)]
            a0 = ac_v[pl.ds(k0, _L)]
            bd = c1 - c0 + a0
            valid = kvec < (cnt - 1)
            term = jnp.where(valid, jnp.abs(bd - 1.0), 1.0)
            term = jnp.where(kvec < tlw, term, 0.0)
            row_v[pl.ds(k0, _L)] = term

        n_w = jnp.where(isum > 0, cnt - 1, 1)
        row_v[pl.ds(_K, _L)] = jnp.full((_L,), n_w, jnp.int32).astype(jnp.float32)
        pltpu.sync_copy(row_v, out_hbm.at[w])


def _tc_finalize(rows_ref, out_ref):
    x = rows_ref[...]
    maxn = jnp.max(x[:, _K:])
    kvec = lax.broadcasted_iota(jnp.int32, (_B, _KCAP), 1).astype(jnp.float32)
    m = (kvec < maxn) & (kvec < float(_K))
    out_ref[0, 0] = jnp.sum(jnp.where(m, x, 0.0)) * (1.0 / _B)


def kernel(alpha, ctc_log_probs, mask, text_length):
    spike, csum = pl.pallas_call(
        _tc_prep,
        grid=(_B,),
        in_specs=[
            pl.BlockSpec((1, _T, 128), lambda b: (b, 0, 0)),
            pl.BlockSpec((1, 1, _T), lambda b: (b, 0, 0)),
            pl.BlockSpec((1, 1, _T), lambda b: (b, 0, 0)),
        ],
        out_specs=[
            pl.BlockSpec((1, 1, _T), lambda b: (b, 0, 0)),
            pl.BlockSpec((1, 1, _T), lambda b: (b, 0, 0)),
        ],
        out_shape=[
            jax.ShapeDtypeStruct((_B, 1, _T), jnp.float32),
            jax.ShapeDtypeStruct((_B, 1, _T), jnp.float32),
        ],
    )(ctc_log_probs, alpha.reshape(_B, 1, _T), mask.reshape(_B, 1, _T))
    spike = spike.reshape(_B, _T)
    csum = csum.reshape(_B, _T)

    mesh = plsc.VectorSubcoreMesh(core_axis_name="c", subcore_axis_name="s")
    sc_run = pl.kernel(
        _sc_body,
        out_type=jax.ShapeDtypeStruct((_B, _KCAP), jnp.float32),
        mesh=mesh,
        compiler_params=pltpu.CompilerParams(needs_layout_passes=False),
        scratch_types=[
            pltpu.VMEM((_T,), jnp.float32),
            pltpu.VMEM((_T,), jnp.float32),
            pltpu.VMEM((_T,), jnp.float32),
            pltpu.VMEM((_KCAP,), jnp.float32),
            pltpu.VMEM((_KCAP,), jnp.float32),
            pltpu.VMEM((_L,), jnp.int32),
            pltpu.VMEM((_KCAP,), jnp.float32),
        ],
    )
    rows = sc_run(spike, csum, alpha, text_length.astype(jnp.int32))

    out = pl.pallas_call(
        _tc_finalize,
        out_shape=jax.ShapeDtypeStruct((1, 1), jnp.float32),
        out_specs=pl.BlockSpec(memory_space=pltpu.SMEM),
    )(rows)
    return out[0, 0]


# speedup vs baseline: 2.0972x; 1.0669x over previous
import math

import jax
import jax.numpy as jnp
from jax import lax
from jax.experimental import pallas as pl
from jax.experimental.pallas import tpu as pltpu
from jax.experimental.pallas import tpu_sc as plsc

_B = 16
_T = 2048
_V = 512
_L = 16
_NCHUNK = _T // _L
_K = 208
_KCAP = 224
_NK = _K // _L
_THR = math.log(1.0 - 0.7)


def _tc_prep(ctc_ref, alpha_ref, mask_ref, spike_ref, csum_ref):
    x2d = ctc_ref[0]
    sp01 = (x2d < _THR).astype(jnp.float32)
    e0 = (lax.broadcasted_iota(jnp.int32, (1, 128), 1) == 0).astype(jnp.float32)
    srow = lax.dot_general(
        e0, sp01, (((1,), (1,)), ((), ()))).reshape(1, 1, _T)
    spike = (srow != 0.0) & (mask_ref[...] != 0.0)
    spike_ref[...] = spike.astype(jnp.float32)
    x = alpha_ref[...]
    sh = 1
    while sh < _T:
        x = x + jnp.concatenate(
            [jnp.zeros((1, 1, sh), jnp.float32), x[:, :, :-sh]], axis=2)
        sh *= 2
    csum_ref[...] = x


def _sc_body(spike_hbm, csum_hbm, alpha_hbm, tl_hbm, out_hbm,
             s_v, c_v, a_v, cc_v, ac_v, tl_v, row_v):
    c = lax.axis_index("c")
    s = lax.axis_index("s")

    @pl.when(c == 0)
    def _work():
        w = s
        pltpu.sync_copy(spike_hbm.at[w], s_v)
        pltpu.sync_copy(csum_hbm.at[w], c_v)
        pltpu.sync_copy(alpha_hbm.at[w], a_v)
        pltpu.sync_copy(tl_hbm, tl_v)

        def zinit(i, _):
            cc_v[pl.ds(i * _L, _L)] = jnp.zeros((_L,), jnp.float32)
            ac_v[pl.ds(i * _L, _L)] = jnp.zeros((_L,), jnp.float32)
            return 0
        lax.fori_loop(0, _KCAP // _L, zinit, 0)

        def chunk(i, carry):
            cnt, ivec = carry
            t0 = i * _L
            spike = s_v[pl.ds(t0, _L)] != 0.0
            spk_i = spike.astype(jnp.int32)
            rank = plsc.cumsum(spk_i)
            pos = cnt + rank - 1
            wmask = spike & (pos < jnp.int32(_KCAP - _L + 8))
            plsc.store_scatter(cc_v, [pos], c_v[pl.ds(t0, _L)], mask=wmask)
            plsc.store_scatter(ac_v, [pos], a_v[pl.ds(t0, _L)], mask=wmask)
            tvec = t0 + lax.iota(jnp.int32, _L)
            ivec = ivec + jnp.where(spike, tvec, 0)
            cnt = cnt + rank[_L - 1]
            return cnt, ivec

        cnt, ivec = lax.fori_loop(
            0, _NCHUNK, chunk,
            (jnp.int32(0), jnp.zeros((_L,), jnp.int32)))
        isum = jnp.sum(ivec)

        tlw = plsc.load_gather(tl_v, [jnp.full((_L,), w, jnp.int32)])
        for kc in range(_NK):
            k0 = kc * _L
            kvec = k0 + lax.iota(jnp.int32, _L)
            c0 = cc_v[pl.ds(k0, _L)]
            c1 = cc_v[pl.ds(k0 + 1, _L)]
            a0 = ac_v[pl.ds(k0, _L)]
            bd = c1 - c0 + a0
            valid = kvec < (cnt - 1)
            term = jnp.where(valid, jnp.abs(bd - 1.0), 1.0)
            term = jnp.where(kvec < tlw, term, 0.0)
            row_v[pl.ds(k0, _L)] = term

        n_w = jnp.where(isum > 0, cnt - 1, 1)
        row_v[pl.ds(_K, _L)] = jnp.full((_L,), n_w, jnp.int32).astype(jnp.float32)
        pltpu.sync_copy(row_v, out_hbm.at[w])


def _tc_finalize(rows_ref, out_ref):
    x = rows_ref[...]
    maxn = jnp.max(x[:, _K:])
    kvec = lax.broadcasted_iota(jnp.int32, (_B, _KCAP), 1).astype(jnp.float32)
    m = (kvec < maxn) & (kvec < float(_K))
    out_ref[0, 0] = jnp.sum(jnp.where(m, x, 0.0)) * (1.0 / _B)


def kernel(alpha, ctc_log_probs, mask, text_length):
    spike, csum = pl.pallas_call(
        _tc_prep,
        grid=(_B,),
        in_specs=[
            pl.BlockSpec((1, _T, 128), lambda b: (b, 0, 0)),
            pl.BlockSpec((1, 1, _T), lambda b: (b, 0, 0)),
            pl.BlockSpec((1, 1, _T), lambda b: (b, 0, 0)),
        ],
        out_specs=[
            pl.BlockSpec((1, 1, _T), lambda b: (b, 0, 0)),
            pl.BlockSpec((1, 1, _T), lambda b: (b, 0, 0)),
        ],
        out_shape=[
            jax.ShapeDtypeStruct((_B, 1, _T), jnp.float32),
            jax.ShapeDtypeStruct((_B, 1, _T), jnp.float32),
        ],
    )(ctc_log_probs, alpha.reshape(_B, 1, _T), mask.reshape(_B, 1, _T))
    spike = spike.reshape(_B, _T)
    csum = csum.reshape(_B, _T)

    mesh = plsc.VectorSubcoreMesh(core_axis_name="c", subcore_axis_name="s")
    sc_run = pl.kernel(
        _sc_body,
        out_type=jax.ShapeDtypeStruct((_B, _KCAP), jnp.float32),
        mesh=mesh,
        compiler_params=pltpu.CompilerParams(needs_layout_passes=False),
        scratch_types=[
            pltpu.VMEM((_T,), jnp.float32),
            pltpu.VMEM((_T,), jnp.float32),
            pltpu.VMEM((_T,), jnp.float32),
            pltpu.VMEM((_KCAP,), jnp.float32),
            pltpu.VMEM((_KCAP,), jnp.float32),
            pltpu.VMEM((_L,), jnp.int32),
            pltpu.VMEM((_KCAP,), jnp.float32),
        ],
    )
    rows = sc_run(spike, csum, alpha, text_length.astype(jnp.int32))

    out = pl.pallas_call(
        _tc_finalize,
        out_shape=jax.ShapeDtypeStruct((1, 1), jnp.float32),
        out_specs=pl.BlockSpec(memory_space=pltpu.SMEM),
    )(rows)
    return out[0, 0]


# speedup vs baseline: 2.1244x; 1.0130x over previous
import math

import jax
import jax.numpy as jnp
from jax import lax
from jax.experimental import pallas as pl
from jax.experimental.pallas import tpu as pltpu
from jax.experimental.pallas import tpu_sc as plsc

_B = 16
_T = 2048
_V = 512
_L = 16
_NCHUNK = _T // _L
_K = 208
_KCAP = 224
_NK = _K // _L
_THR = math.log(1.0 - 0.7)


def _tc_prep(ctc_ref, alpha_ref, mask_ref, spike_ref, csum_ref):
    x2d = ctc_ref[0]
    sp01 = (x2d < _THR).astype(jnp.float32)
    e0 = (lax.broadcasted_iota(jnp.int32, (1, 128), 1) == 0).astype(jnp.float32)
    srow = lax.dot_general(
        e0, sp01, (((1,), (1,)), ((), ()))).reshape(1, 1, _T)
    spike = (srow != 0.0) & (mask_ref[...] != 0.0)
    spike_ref[...] = spike.astype(jnp.float32)
    x = alpha_ref[...]
    sh = 1
    while sh < _T:
        x = x + jnp.concatenate(
            [jnp.zeros((1, 1, sh), jnp.float32), x[:, :, :-sh]], axis=2)
        sh *= 2
    csum_ref[...] = x


def _sc_body(spike_hbm, csum_hbm, alpha_hbm, tl_hbm, out_hbm,
             s_v, c_v, a_v, cc_v, ac_v, tl_v, row_v):
    c = lax.axis_index("c")
    s = lax.axis_index("s")

    @pl.when(c == 0)
    def _work():
        w = s
        pltpu.sync_copy(spike_hbm.at[w], s_v)
        pltpu.sync_copy(csum_hbm.at[w], c_v)
        pltpu.sync_copy(alpha_hbm.at[w], a_v)
        pltpu.sync_copy(tl_hbm, tl_v)

        def zinit(i, _):
            cc_v[pl.ds(i * _L, _L)] = jnp.zeros((_L,), jnp.float32)
            ac_v[pl.ds(i * _L, _L)] = jnp.zeros((_L,), jnp.float32)
            return 0
        lax.fori_loop(0, _KCAP // _L, zinit, 0)

        def chunk(i, carry):
            cnt, ivec = carry
            t0 = i * _L
            spike = s_v[pl.ds(t0, _L)] != 0.0
            off = jnp.minimum(cnt, jnp.int32(_K))
            plsc.store_compressed(cc_v.at[pl.ds(off, _L)],
                                  c_v[pl.ds(t0, _L)], mask=spike)
            plsc.store_compressed(ac_v.at[pl.ds(off, _L)],
                                  a_v[pl.ds(t0, _L)], mask=spike)
            tvec = t0 + lax.iota(jnp.int32, _L)
            ivec = ivec + jnp.where(spike, tvec, 0)
            cnt = cnt + plsc.all_reduce_population_count(spike)[0]
            return cnt, ivec

        cnt, ivec = lax.fori_loop(
            0, _NCHUNK, chunk,
            (jnp.int32(0), jnp.zeros((_L,), jnp.int32)), unroll=4)
        isum = jnp.sum(ivec)

        tlw = plsc.load_gather(tl_v, [jnp.full((_L,), w, jnp.int32)])
        for kc in range(_NK):
            k0 = kc * _L
            kvec = k0 + lax.iota(jnp.int32, _L)
            c0 = cc_v[pl.ds(k0, _L)]
            c1 = cc_v[pl.ds(k0 + 1, _L)]
            a0 = ac_v[pl.ds(k0, _L)]
            bd = c1 - c0 + a0
            valid = kvec < (cnt - 1)
            term = jnp.where(valid, jnp.abs(bd - 1.0), 1.0)
            term = jnp.where(kvec < tlw, term, 0.0)
            row_v[pl.ds(k0, _L)] = term

        n_w = jnp.where(isum > 0, cnt - 1, 1)
        row_v[pl.ds(_K, _L)] = jnp.full((_L,), n_w, jnp.int32).astype(jnp.float32)
        pltpu.sync_copy(row_v, out_hbm.at[w])


def _tc_finalize(rows_ref, out_ref):
    x = rows_ref[...]
    maxn = jnp.max(x[:, _K:])
    kvec = lax.broadcasted_iota(jnp.int32, (_B, _KCAP), 1).astype(jnp.float32)
    m = (kvec < maxn) & (kvec < float(_K))
    out_ref[0, 0] = jnp.sum(jnp.where(m, x, 0.0)) * (1.0 / _B)


def kernel(alpha, ctc_log_probs, mask, text_length):
    spike, csum = pl.pallas_call(
        _tc_prep,
        grid=(_B,),
        in_specs=[
            pl.BlockSpec((1, _T, 128), lambda b: (b, 0, 0)),
            pl.BlockSpec((1, 1, _T), lambda b: (b, 0, 0)),
            pl.BlockSpec((1, 1, _T), lambda b: (b, 0, 0)),
        ],
        out_specs=[
            pl.BlockSpec((1, 1, _T), lambda b: (b, 0, 0)),
            pl.BlockSpec((1, 1, _T), lambda b: (b, 0, 0)),
        ],
        out_shape=[
            jax.ShapeDtypeStruct((_B, 1, _T), jnp.float32),
            jax.ShapeDtypeStruct((_B, 1, _T), jnp.float32),
        ],
    )(ctc_log_probs, alpha.reshape(_B, 1, _T), mask.reshape(_B, 1, _T))
    spike = spike.reshape(_B, _T)
    csum = csum.reshape(_B, _T)

    mesh = plsc.VectorSubcoreMesh(core_axis_name="c", subcore_axis_name="s")
    sc_run = pl.kernel(
        _sc_body,
        out_type=jax.ShapeDtypeStruct((_B, _KCAP), jnp.float32),
        mesh=mesh,
        compiler_params=pltpu.CompilerParams(needs_layout_passes=False),
        scratch_types=[
            pltpu.VMEM((_T,), jnp.float32),
            pltpu.VMEM((_T,), jnp.float32),
            pltpu.VMEM((_T,), jnp.float32),
            pltpu.VMEM((_KCAP,), jnp.float32),
            pltpu.VMEM((_KCAP,), jnp.float32),
            pltpu.VMEM((_L,), jnp.int32),
            pltpu.VMEM((_KCAP,), jnp.float32),
        ],
    )
    rows = sc_run(spike, csum, alpha, text_length.astype(jnp.int32))

    out = pl.pallas_call(
        _tc_finalize,
        out_shape=jax.ShapeDtypeStruct((1, 1), jnp.float32),
        out_specs=pl.BlockSpec(memory_space=pltpu.SMEM),
    )(rows)
    return out[0, 0]


# speedup vs baseline: 2.4715x; 1.1634x over previous
import math

import jax
import jax.numpy as jnp
from jax import lax
from jax.experimental import pallas as pl
from jax.experimental.pallas import tpu as pltpu
from jax.experimental.pallas import tpu_sc as plsc

_B = 16
_T = 2048
_V = 512
_L = 16
_NCHUNK = _T // _L
_K = 208
_KCAP = 224
_NK = _K // _L
_THR = math.log(1.0 - 0.7)


def _tc_prep(ctc_ref, alpha_ref, mask_ref, spike_ref, csum_ref, excl_ref):
    r = pl.program_id(0) % 8
    x2d = ctc_ref[0]
    sp01 = (x2d < _THR).astype(jnp.float32)
    e0 = (lax.broadcasted_iota(jnp.int32, (1, 128), 1) == 0).astype(jnp.float32)
    srow = lax.dot_general(
        e0, sp01, (((1,), (1,)), ((), ()))).reshape(1, 1, _T)
    arow = alpha_ref[pl.ds(r, 1), :].reshape(1, 1, _T)
    mrow = mask_ref[pl.ds(r, 1), :].reshape(1, 1, _T)
    spike = (srow != 0.0) & (mrow != 0.0)
    spike_ref[...] = spike.astype(jnp.float32)
    x = arow
    sh = 1
    while sh < _T:
        x = x + jnp.concatenate(
            [jnp.zeros((1, 1, sh), jnp.float32), x[:, :, :-sh]], axis=2)
        sh *= 2
    csum_ref[...] = x
    excl_ref[...] = x - arow


def _sc_body(spike_hbm, csum_hbm, excl_hbm, tl_hbm, out_hbm,
             s_v, c_v, e_v, cc_v, ec_v, tl_v, row_v):
    w = lax.axis_index("s")

    pltpu.sync_copy(spike_hbm.at[w, 0], s_v)
    pltpu.sync_copy(csum_hbm.at[w, 0], c_v)
    pltpu.sync_copy(excl_hbm.at[w, 0], e_v)
    pltpu.sync_copy(tl_hbm, tl_v)

    def zinit(i, _):
        cc_v[pl.ds(i * _L, _L)] = jnp.zeros((_L,), jnp.float32)
        ec_v[pl.ds(i * _L, _L)] = jnp.zeros((_L,), jnp.float32)
        return 0
    lax.fori_loop(0, _KCAP // _L, zinit, 0)

    def chunk(i, carry):
        cnt, ivec = carry
        t0 = i * _L
        spike = s_v[pl.ds(t0, _L)] != 0.0
        off = jnp.minimum(cnt, jnp.int32(_K))
        plsc.store_compressed(cc_v.at[pl.ds(off, _L)],
                              c_v[pl.ds(t0, _L)], mask=spike)
        plsc.store_compressed(ec_v.at[pl.ds(off, _L)],
                              e_v[pl.ds(t0, _L)], mask=spike)
        tvec = t0 + lax.iota(jnp.int32, _L)
        ivec = ivec + jnp.where(spike, tvec, 0)
        cnt = cnt + plsc.all_reduce_population_count(spike)[0]
        return cnt, ivec

    cnt, ivec = lax.fori_loop(
        0, _NCHUNK, chunk,
        (jnp.int32(0), jnp.zeros((_L,), jnp.int32)), unroll=4)
    isum = jnp.sum(ivec)

    tlw = plsc.load_gather(tl_v, [jnp.full((_L,), w, jnp.int32)])
    for kc in range(_NK):
        k0 = kc * _L
        kvec = k0 + lax.iota(jnp.int32, _L)
        c1 = cc_v[pl.ds(k0 + 1, _L)]
        e0 = ec_v[pl.ds(k0, _L)]
        bd = c1 - e0
        valid = kvec < (cnt - 1)
        term = jnp.where(valid, jnp.abs(bd - 1.0), 1.0)
        term = jnp.where(kvec < tlw, term, 0.0)
        row_v[pl.ds(k0, _L)] = term

    n_w = jnp.where(isum > 0, cnt - 1, 1)
    row_v[pl.ds(_K, _L)] = jnp.full((_L,), n_w, jnp.int32).astype(jnp.float32)
    pltpu.sync_copy(row_v, out_hbm.at[w])


def _tc_finalize(rows_ref, out_ref):
    x = rows_ref[...]
    maxn = jnp.max(x[:, _K:])
    kvec = lax.broadcasted_iota(jnp.int32, (_B, _KCAP), 1).astype(jnp.float32)
    m = (kvec < maxn) & (kvec < float(_K))
    out_ref[0, 0] = jnp.sum(jnp.where(m, x, 0.0)) * (1.0 / _B)


def kernel(alpha, ctc_log_probs, mask, text_length):
    spike, csum, excl = pl.pallas_call(
        _tc_prep,
        grid=(_B,),
        in_specs=[
            pl.BlockSpec((1, _T, 128), lambda b: (b, 0, 0)),
            pl.BlockSpec((8, _T), lambda b: (b // 8, 0)),
            pl.BlockSpec((8, _T), lambda b: (b // 8, 0)),
        ],
        out_specs=[
            pl.BlockSpec((1, 1, _T), lambda b: (b, 0, 0)),
            pl.BlockSpec((1, 1, _T), lambda b: (b, 0, 0)),
            pl.BlockSpec((1, 1, _T), lambda b: (b, 0, 0)),
        ],
        out_shape=[
            jax.ShapeDtypeStruct((_B, 1, _T), jnp.float32),
            jax.ShapeDtypeStruct((_B, 1, _T), jnp.float32),
            jax.ShapeDtypeStruct((_B, 1, _T), jnp.float32),
        ],
    )(ctc_log_probs, alpha, mask)

    mesh = plsc.VectorSubcoreMesh(
        core_axis_name="c", subcore_axis_name="s", num_cores=1)
    sc_run = pl.kernel(
        _sc_body,
        out_type=jax.ShapeDtypeStruct((_B, _KCAP), jnp.float32),
        mesh=mesh,
        compiler_params=pltpu.CompilerParams(needs_layout_passes=False),
        scratch_types=[
            pltpu.VMEM((_T,), jnp.float32),
            pltpu.VMEM((_T,), jnp.float32),
            pltpu.VMEM((_T,), jnp.float32),
            pltpu.VMEM((_KCAP,), jnp.float32),
            pltpu.VMEM((_KCAP,), jnp.float32),
            pltpu.VMEM((_L,), jnp.int32),
            pltpu.VMEM((_KCAP,), jnp.float32),
        ],
    )
    rows = sc_run(spike, csum, excl, text_length.astype(jnp.int32))

    out = pl.pallas_call(
        _tc_finalize,
        out_shape=jax.ShapeDtypeStruct((1, 1), jnp.float32),
        out_specs=pl.BlockSpec(memory_space=pltpu.SMEM),
    )(rows)
    return out[0, 0]


# speedup vs baseline: 2.5658x; 1.0381x over previous
import math

import jax
import jax.numpy as jnp
from jax import lax
from jax.experimental import pallas as pl
from jax.experimental.pallas import tpu as pltpu
from jax.experimental.pallas import tpu_sc as plsc

_B = 16
_T = 2048
_V = 512
_L = 16
_NCHUNK = _T // _L
_K = 208
_KCAP = 224
_NK = _K // _L
_THR = math.log(1.0 - 0.7)


def _tc_prep(ctc_ref, alpha_ref, mask_ref, spike_ref, csum_ref, excl_ref):
    r = pl.program_id(0) % 8
    x2d = ctc_ref[0]
    sp01 = (x2d < _THR).astype(jnp.float32)
    e0 = (lax.broadcasted_iota(jnp.int32, (1, 128), 1) == 0).astype(jnp.float32)
    srow = lax.dot_general(
        e0, sp01, (((1,), (1,)), ((), ()))).reshape(1, 1, _T)
    arow = alpha_ref[pl.ds(r, 1), :].reshape(1, 1, _T)
    mrow = mask_ref[pl.ds(r, 1), :].reshape(1, 1, _T)
    spike = (srow != 0.0) & (mrow != 0.0)
    spike_ref[...] = spike.astype(jnp.float32)
    x = arow
    sh = 1
    while sh < _T:
        x = x + jnp.concatenate(
            [jnp.zeros((1, 1, sh), jnp.float32), x[:, :, :-sh]], axis=2)
        sh *= 2
    csum_ref[...] = x
    excl_ref[...] = x - arow


def _sc_body(spike_hbm, csum_hbm, excl_hbm, out_hbm,
             s_v, c_v, e_v, cc_v, ec_v, row_v, sem):
    w = lax.axis_index("s")

    cps = [
        pltpu.make_async_copy(spike_hbm.at[w, 0], s_v, sem),
        pltpu.make_async_copy(csum_hbm.at[w, 0], c_v, sem),
        pltpu.make_async_copy(excl_hbm.at[w, 0], e_v, sem),
    ]
    for cp in cps:
        cp.start()

    def zinit(i, _):
        cc_v[pl.ds(i * _L, _L)] = jnp.zeros((_L,), jnp.float32)
        ec_v[pl.ds(i * _L, _L)] = jnp.zeros((_L,), jnp.float32)
        return 0
    lax.fori_loop(0, _KCAP // _L, zinit, 0)
    for cp in cps:
        cp.wait()

    def chunk(i, cnt):
        t0 = i * _L
        spike = s_v[pl.ds(t0, _L)] != 0.0
        off = jnp.minimum(cnt, jnp.int32(_K))
        plsc.store_compressed(cc_v.at[pl.ds(off, _L)],
                              c_v[pl.ds(t0, _L)], mask=spike)
        plsc.store_compressed(ec_v.at[pl.ds(off, _L)],
                              e_v[pl.ds(t0, _L)], mask=spike)
        return cnt + plsc.all_reduce_population_count(spike)[0]

    cnt = lax.fori_loop(0, _NCHUNK, chunk, jnp.int32(0), unroll=8)

    for kc in range(_NK):
        k0 = kc * _L
        kvec = k0 + lax.iota(jnp.int32, _L)
        c1 = cc_v[pl.ds(k0 + 1, _L)]
        e0 = ec_v[pl.ds(k0, _L)]
        valid = kvec < (cnt - 1)
        row_v[pl.ds(k0, _L)] = jnp.where(
            valid, jnp.abs(c1 - e0 - 1.0), 1.0)

    row_v[pl.ds(_K, _L)] = jnp.full((_L,), cnt, jnp.int32).astype(jnp.float32)
    pltpu.sync_copy(row_v, out_hbm.at[w])


def _tc_finalize(rows_ref, spike_ref, tl_ref, out_ref):
    x = rows_ref[...]
    cnt = x[:, _K:_K + 1]
    sp = spike_ref[...].reshape(_B, _T)
    tpos = lax.broadcasted_iota(jnp.int32, (_B, _T), 1).astype(jnp.float32)
    isum = jnp.sum(sp * tpos, axis=1, keepdims=True)
    n = jnp.where(isum > 0.0, cnt - 1.0, 1.0)
    maxn = jnp.max(n)
    tl = tl_ref[...].astype(jnp.float32)
    kvec = lax.broadcasted_iota(jnp.int32, (_B, _KCAP), 1).astype(jnp.float32)
    m = (kvec < maxn) & (kvec < tl) & (kvec < float(_K))
    out_ref[0, 0] = jnp.sum(jnp.where(m, x, 0.0)) * (1.0 / _B)


def kernel(alpha, ctc_log_probs, mask, text_length):
    spike, csum, excl = pl.pallas_call(
        _tc_prep,
        grid=(_B,),
        in_specs=[
            pl.BlockSpec((1, _T, 128), lambda b: (b, 0, 0)),
            pl.BlockSpec((8, _T), lambda b: (b // 8, 0)),
            pl.BlockSpec((8, _T), lambda b: (b // 8, 0)),
        ],
        out_specs=[
            pl.BlockSpec((1, 1, _T), lambda b: (b, 0, 0)),
            pl.BlockSpec((1, 1, _T), lambda b: (b, 0, 0)),
            pl.BlockSpec((1, 1, _T), lambda b: (b, 0, 0)),
        ],
        out_shape=[
            jax.ShapeDtypeStruct((_B, 1, _T), jnp.float32),
            jax.ShapeDtypeStruct((_B, 1, _T), jnp.float32),
            jax.ShapeDtypeStruct((_B, 1, _T), jnp.float32),
        ],
    )(ctc_log_probs, alpha, mask)

    mesh = plsc.VectorSubcoreMesh(
        core_axis_name="c", subcore_axis_name="s", num_cores=1)
    sc_run = pl.kernel(
        _sc_body,
        out_type=jax.ShapeDtypeStruct((_B, _KCAP), jnp.float32),
        mesh=mesh,
        compiler_params=pltpu.CompilerParams(needs_layout_passes=False),
        scratch_types=[
            pltpu.VMEM((_T,), jnp.float32),
            pltpu.VMEM((_T,), jnp.float32),
            pltpu.VMEM((_T,), jnp.float32),
            pltpu.VMEM((_KCAP,), jnp.float32),
            pltpu.VMEM((_KCAP,), jnp.float32),
            pltpu.VMEM((_KCAP,), jnp.float32),
            pltpu.SemaphoreType.DMA,
        ],
    )
    rows = sc_run(spike, csum, excl)

    out = pl.pallas_call(
        _tc_finalize,
        out_shape=jax.ShapeDtypeStruct((1, 1), jnp.float32),
        out_specs=pl.BlockSpec(memory_space=pltpu.SMEM),
    )(rows, spike, text_length.astype(jnp.int32).reshape(_B, 1))
    return out[0, 0]
